# baseline (speedup 1.0000x reference)
import functools

import jax
import jax.numpy as jnp
from jax.experimental import pallas as pl
from jax.experimental.pallas import tpu as pltpu

ALPHA = 0.2
SPAD = 8


def _gat_kernel(h_ref, adjl_ref, adjr_ref, w_ref, ap_ref, out_ref, hw_ref,
                s_ref, *, ti, nh):
    i = pl.program_id(1)

    @pl.when(i == 0)
    def _project():
        hw = jnp.dot(h_ref[0], w_ref[...], preferred_element_type=jnp.float32)
        hw_ref[...] = hw.astype(jnp.bfloat16)
        s_ref[...] = jnp.dot(hw, ap_ref[...],
                             preferred_element_type=jnp.float32)

    si = s_ref[pl.ds(i * ti, ti), 0:1]
    sjl = s_ref[:nh, 1:2]
    sjr = s_ref[nh:, 1:2]
    xl = si + sjl.T
    xr = si + sjr.T
    el = jnp.exp2(jnp.minimum(xl, ALPHA * xl)) * adjl_ref[0]
    er = jnp.exp2(jnp.minimum(xr, ALPHA * xr)) * adjr_ref[0]
    rowsum = (jnp.sum(el, axis=1, keepdims=True)
              + jnp.sum(er, axis=1, keepdims=True))
    denom = jnp.where(rowsum != 0, rowsum, 1.0)
    prod = (jnp.dot(el.astype(jnp.bfloat16), hw_ref[:nh],
                    preferred_element_type=jnp.float32)
            + jnp.dot(er.astype(jnp.bfloat16), hw_ref[nh:],
                      preferred_element_type=jnp.float32))
    hp = prod / denom
    out_ref[0] = jnp.where(hp > 0, hp, jnp.exp(hp) - 1.0)


@jax.jit
def kernel(h, adj, W, a):
    B, N, D = h.shape
    a1 = a[0, :D]
    a2 = a[0, D:]
    neg_log2e = -1.4426950408889634
    a_pair = jnp.zeros((D, SPAD), jnp.float32)
    a_pair = (a_pair
              .at[:, 0].set(neg_log2e * a1)
              .at[:, 1].set(neg_log2e * a2))

    TI = 1024
    ni = N // TI
    out = pl.pallas_call(
        functools.partial(_gat_kernel, ti=TI, nh=N // 2),
        grid=(B, ni),
        in_specs=[
            pl.BlockSpec((1, N, D), lambda b, i: (b, 0, 0)),
            pl.BlockSpec((1, TI, N // 2), lambda b, i: (b, i, 0)),
            pl.BlockSpec((1, TI, N // 2), lambda b, i: (b, i, 1)),
            pl.BlockSpec((D, D), lambda b, i: (0, 0)),
            pl.BlockSpec((D, SPAD), lambda b, i: (0, 0)),
        ],
        out_specs=pl.BlockSpec((1, TI, D), lambda b, i: (b, i, 0)),
        out_shape=jax.ShapeDtypeStruct((B, N, D), jnp.float32),
        scratch_shapes=[
            pltpu.VMEM((N, D), jnp.bfloat16),
            pltpu.VMEM((N, SPAD), jnp.float32),
        ],
        compiler_params=pltpu.CompilerParams(
            dimension_semantics=("parallel", "arbitrary"),
            vmem_limit_bytes=110 * 1024 * 1024,
        ),
    )(h, adj, adj, W, a_pair)
    return out


# speedup vs baseline: 1.0238x; 1.0238x over previous
import functools

import jax
import jax.numpy as jnp
from jax.experimental import pallas as pl
from jax.experimental.pallas import tpu as pltpu

ALPHA = 0.2
SPAD = 8


def _gat_kernel(h_ref, adj_ref, w_ref, ap_ref, out_ref, hw_ref, s_ref, *, ti):
    i = pl.program_id(1)

    @pl.when(i == 0)
    def _project():
        hw = jnp.dot(h_ref[0], w_ref[...], preferred_element_type=jnp.float32)
        hw_ref[...] = hw.astype(jnp.bfloat16)
        s_ref[...] = jnp.dot(hw, ap_ref[...],
                             preferred_element_type=jnp.float32)

    si = s_ref[pl.ds(i * ti, ti), 0:1]
    sj = s_ref[:, 1:2]
    x = si + sj.T
    e = jnp.exp2(jnp.minimum(x, ALPHA * x)) * adj_ref[0]
    rowsum = jnp.sum(e, axis=1, keepdims=True)
    denom = jnp.where(rowsum != 0, rowsum, 1.0)
    prod = jnp.dot(e.astype(jnp.bfloat16), hw_ref[...],
                   preferred_element_type=jnp.float32)
    hp = prod / denom
    out_ref[0] = jnp.where(hp > 0, hp, jnp.exp(hp) - 1.0)


@jax.jit
def kernel(h, adj, W, a):
    B, N, D = h.shape
    a1 = a[0, :D]
    a2 = a[0, D:]
    neg_log2e = -1.4426950408889634
    a_pair = jnp.zeros((D, SPAD), jnp.float32)
    a_pair = (a_pair
              .at[:, 0].set(neg_log2e * a1)
              .at[:, 1].set(neg_log2e * a2))

    TI = 1024
    ni = N // TI
    out = pl.pallas_call(
        functools.partial(_gat_kernel, ti=TI),
        grid=(B, ni),
        in_specs=[
            pl.BlockSpec((1, N, D), lambda b, i: (b, 0, 0)),
            pl.BlockSpec((1, TI, N), lambda b, i: (b, i, 0)),
            pl.BlockSpec((D, D), lambda b, i: (0, 0)),
            pl.BlockSpec((D, SPAD), lambda b, i: (0, 0)),
        ],
        out_specs=pl.BlockSpec((1, TI, D), lambda b, i: (b, i, 0)),
        out_shape=jax.ShapeDtypeStruct((B, N, D), jnp.float32),
        scratch_shapes=[
            pltpu.VMEM((N, D), jnp.bfloat16),
            pltpu.VMEM((N, SPAD), jnp.float32),
        ],
        compiler_params=pltpu.CompilerParams(
            dimension_semantics=("parallel", "arbitrary"),
            vmem_limit_bytes=110 * 1024 * 1024,
        ),
    )(h, adj, W, a_pair)
    return out
